# Initial kernel scaffold; baseline (speedup 1.0000x reference)
import jax
import jax.numpy as jnp
from jax.experimental import pallas as pl


def kernel(inp, table):
    raise NotImplementedError("write your pallas kernel here")


# speedup vs baseline: 1.4611x; 1.4611x over previous
import functools

import jax
import jax.numpy as jnp
from jax import lax
from jax.experimental import pallas as pl
from jax.experimental.pallas import tpu as pltpu
from jax.experimental.pallas import tpu_sc as plsc

B = 4096
L = 200
DIM = 32
N = B * L
IDX_MINOR = 128
ROWS = N // IDX_MINOR
NC, NS = 2, 16
NW = NC * NS
ROWS_W = ROWS // NW
G = 8
NG = ROWS_W // G
CHUNK = G * IDX_MINOR


def _body(idx_hbm, table_hbm, out_hbm, idx_v, rows_v, sem):
    wid = lax.axis_index("s") * NC + lax.axis_index("c")
    row_base = wid * ROWS_W

    def group(g, carry):
        roff = row_base + g * G
        pltpu.sync_copy(idx_hbm.at[pl.ds(roff, G)], idx_v)
        copies = [
            pltpu.async_copy(
                table_hbm.at[idx_v.at[j]],
                rows_v.at[pl.ds(j * IDX_MINOR, IDX_MINOR)],
                sem,
            )
            for j in range(G)
        ]
        for cp in copies:
            cp.wait()
        pltpu.sync_copy(rows_v, out_hbm.at[pl.ds(roff * IDX_MINOR, CHUNK)])
        return carry

    lax.fori_loop(0, NG, group, 0)


@jax.jit
def kernel(inp, table):
    idx = inp.reshape(ROWS, IDX_MINOR).astype(jnp.int32)
    out = pl.kernel(
        _body,
        out_type=jax.ShapeDtypeStruct((N, DIM), jnp.float32),
        mesh=plsc.VectorSubcoreMesh(core_axis_name="c", subcore_axis_name="s"),
        compiler_params=pltpu.CompilerParams(use_tc_tiling_on_sc=False),
        scratch_types=[
            pltpu.VMEM((G, IDX_MINOR), jnp.int32),
            pltpu.VMEM((CHUNK, DIM), jnp.float32),
            pltpu.SemaphoreType.DMA,
        ],
    )(idx, table)
    return out.reshape(B, L, DIM)


# speedup vs baseline: 1.5020x; 1.0279x over previous
import jax
import jax.numpy as jnp
from jax import lax
from jax.experimental import pallas as pl
from jax.experimental.pallas import tpu as pltpu
from jax.experimental.pallas import tpu_sc as plsc

B = 4096
L = 200
DIM = 32
N = B * L
IDX_MINOR = 128
ROWS = N // IDX_MINOR
NC, NS = 2, 16
NW = NC * NS
ROWS_W = ROWS // NW
G = 2
CHUNK = G * IDX_MINOR
NG = ROWS_W // G
NBUF = 4
SG = NG // NBUF


def _body(idx_hbm, table_hbm, out_hbm, idx_v,
          r0, r1, r2, r3, g0, g1, g2, g3, s0, s1, s2, s3):
    rows = (r0, r1, r2, r3)
    gsem = (g0, g1, g2, g3)
    ssem = (s0, s1, s2, s3)

    wid = lax.axis_index("s") * NC + lax.axis_index("c")
    row_base = wid * ROWS_W

    def fire(g, b):
        for j in range(G):
            pltpu.async_copy(
                table_hbm.at[idx_v.at[g * G + j]],
                rows[b].at[pl.ds(j * IDX_MINOR, IDX_MINOR)],
                gsem[b],
            )

    def wait_gathers(g, b):
        for j in range(G):
            pltpu.make_async_copy(
                table_hbm.at[idx_v.at[g * G + j]],
                rows[b].at[pl.ds(j * IDX_MINOR, IDX_MINOR)],
                gsem[b],
            ).wait()

    def store(g, b):
        pltpu.async_copy(
            rows[b],
            out_hbm.at[pl.ds((row_base + g * G) * IDX_MINOR, CHUNK)],
            ssem[b],
        )

    def wait_store(g, b):
        pltpu.make_async_copy(
            rows[b],
            out_hbm.at[pl.ds((row_base + g * G) * IDX_MINOR, CHUNK)],
            ssem[b],
        ).wait()

    pltpu.sync_copy(idx_hbm.at[pl.ds(row_base, ROWS_W)], idx_v)

    for b in range(NBUF - 1):
        fire(b, b)

    for b in range(NBUF):
        g = b
        wait_gathers(g, b)
        store(g, b)
        b2 = (b + NBUF - 1) % NBUF
        if g >= 1:
            wait_store(g - 1, b2)
        fire(g + NBUF - 1, b2)

    def sbody(s, carry):
        gbase = s * NBUF
        for b in range(NBUF):
            g = gbase + b
            wait_gathers(g, b)
            store(g, b)
            b2 = (b + NBUF - 1) % NBUF
            wait_store(g - 1, b2)
            fire(g + NBUF - 1, b2)
        return carry

    lax.fori_loop(1, SG - 1, sbody, 0)

    for b in range(NBUF):
        g = NG - NBUF + b
        wait_gathers(g, b)
        store(g, b)
        if b == 0:
            b2 = NBUF - 1
            wait_store(g - 1, b2)
            fire(NG - 1, b2)

    for b in range(NBUF):
        wait_store(NG - NBUF + b, b)


@jax.jit
def kernel(inp, table):
    idx = inp.reshape(ROWS, IDX_MINOR).astype(jnp.int32)
    out = pl.kernel(
        _body,
        out_type=jax.ShapeDtypeStruct((N, DIM), jnp.float32),
        mesh=plsc.VectorSubcoreMesh(core_axis_name="c", subcore_axis_name="s"),
        compiler_params=pltpu.CompilerParams(use_tc_tiling_on_sc=False),
        scratch_types=[
            pltpu.VMEM((ROWS_W, IDX_MINOR), jnp.int32),
        ]
        + [pltpu.VMEM((CHUNK, DIM), jnp.float32) for _ in range(NBUF)]
        + [pltpu.SemaphoreType.DMA for _ in range(2 * NBUF)],
    )(idx, table)
    return out.reshape(B, L, DIM)
